# baseline (device time: 61478 ns/iter reference)
import jax
import jax.numpy as jnp
from jax import lax
from jax.experimental import pallas as pl
from jax.experimental.pallas import tpu as pltpu

C = 16


def kernel(x):
    _, m, n = x.shape
    half = m // 2
    ck = half // C

    def body(x_ref, out_ref, xoth, xme, send_buf, rs_recv, mytile, oxtile,
             ld_oth_s, ld_me_s, st_me_s, st_ox_s,
             rs_ss, rs_rs, agy_ss, agy_rs, agx_ss, agx_rs, fwd_ss, fwd_rs):
        my_x = lax.axis_index("x")
        my_y = lax.axis_index("y")
        ox = 1 - my_x
        oy = 1 - my_y
        xp = (ox, my_y)
        yp = (my_x, oy)

        r_me = my_x * half
        r_ot = ox * half
        c_me = my_y * n

        ld_oth, ld_me = [], []
        for k in range(C):
            d = pltpu.make_async_copy(
                x_ref.at[0, pl.ds(r_ot + k * ck, ck), :],
                xoth.at[pl.ds(k * ck, ck), :],
                ld_oth_s.at[k],
            )
            d.start()
            ld_oth.append(d)
        for k in range(C):
            d = pltpu.make_async_copy(
                x_ref.at[0, pl.ds(r_me + k * ck, ck), :],
                xme.at[pl.ds(k * ck, ck), :],
                ld_me_s.at[k],
            )
            d.start()
            ld_me.append(d)

        barrier = pltpu.get_barrier_semaphore()
        for nbr in (xp, yp):
            pl.semaphore_signal(
                barrier, inc=1, device_id=nbr,
                device_id_type=pl.DeviceIdType.MESH,
            )
        pl.semaphore_wait(barrier, 2)

        rs = []
        for k in range(C):
            ld_oth[k].wait()
            send_buf[pl.ds(k * ck, ck), :] = xoth[
                pl.ds(k * ck, ck), :
            ].astype(jnp.bfloat16)
            d = pltpu.make_async_remote_copy(
                src_ref=send_buf.at[pl.ds(k * ck, ck), :],
                dst_ref=rs_recv.at[pl.ds(k * ck, ck), :],
                send_sem=rs_ss.at[k],
                recv_sem=rs_rs.at[k],
                device_id=xp,
                device_id_type=pl.DeviceIdType.MESH,
            )
            d.start()
            rs.append(d)

        ag_y, ag_x, st_me = [], [], []
        for k in range(C):
            rs[k].wait_recv()
            ld_me[k].wait()
            sl = pl.ds(k * ck, ck)
            mytile[sl, :] = (
                xme[sl, :].astype(jnp.bfloat16) + rs_recv[sl, :]
            )
            out_me = out_ref.at[pl.ds(r_me + k * ck, ck), pl.ds(c_me, n)]
            dy = pltpu.make_async_remote_copy(
                src_ref=mytile.at[sl, :],
                dst_ref=out_me,
                send_sem=agy_ss.at[k], recv_sem=agy_rs.at[k],
                device_id=yp, device_id_type=pl.DeviceIdType.MESH,
            )
            dx = pltpu.make_async_remote_copy(
                src_ref=mytile.at[sl, :],
                dst_ref=oxtile.at[sl, :],
                send_sem=agx_ss.at[k], recv_sem=agx_rs.at[k],
                device_id=xp, device_id_type=pl.DeviceIdType.MESH,
            )
            dy.start()
            dx.start()
            ag_y.append(dy)
            ag_x.append(dx)
            st = pltpu.make_async_copy(mytile.at[sl, :], out_me, st_me_s.at[k])
            st.start()
            st_me.append(st)

        fwd, st_ox = [], []
        for k in range(C):
            ag_x[k].wait_recv()
            sl = pl.ds(k * ck, ck)
            out_ox = out_ref.at[pl.ds(r_ot + k * ck, ck), pl.ds(c_me, n)]
            d = pltpu.make_async_remote_copy(
                src_ref=oxtile.at[sl, :],
                dst_ref=out_ox,
                send_sem=fwd_ss.at[k], recv_sem=fwd_rs.at[k],
                device_id=yp, device_id_type=pl.DeviceIdType.MESH,
            )
            d.start()
            fwd.append(d)
            st = pltpu.make_async_copy(oxtile.at[sl, :], out_ox, st_ox_s.at[k])
            st.start()
            st_ox.append(st)

        for k in range(C):
            rs[k].wait_send()
            ag_x[k].wait_send()
            ag_y[k].wait()
            fwd[k].wait()
            st_me[k].wait()
            st_ox[k].wait()

    return pl.pallas_call(
        body,
        out_shape=jax.ShapeDtypeStruct((m, 2 * n), jnp.bfloat16),
        in_specs=[pl.BlockSpec(memory_space=pltpu.MemorySpace.HBM)],
        out_specs=pl.BlockSpec(memory_space=pltpu.MemorySpace.HBM),
        scratch_shapes=[
            pltpu.VMEM((half, n), jnp.float32),
            pltpu.VMEM((half, n), jnp.float32),
            pltpu.VMEM((half, n), jnp.bfloat16),
            pltpu.VMEM((half, n), jnp.bfloat16),
            pltpu.VMEM((half, n), jnp.bfloat16),
            pltpu.VMEM((half, n), jnp.bfloat16),
            pltpu.SemaphoreType.DMA((C,)),
            pltpu.SemaphoreType.DMA((C,)),
            pltpu.SemaphoreType.DMA((C,)),
            pltpu.SemaphoreType.DMA((C,)),
            pltpu.SemaphoreType.DMA((C,)),
            pltpu.SemaphoreType.DMA((C,)),
            pltpu.SemaphoreType.DMA((C,)),
            pltpu.SemaphoreType.DMA((C,)),
            pltpu.SemaphoreType.DMA((C,)),
            pltpu.SemaphoreType.DMA((C,)),
            pltpu.SemaphoreType.DMA((C,)),
            pltpu.SemaphoreType.DMA((C,)),
        ],
        compiler_params=pltpu.CompilerParams(collective_id=0),
    )(x)


# device time: 60247 ns/iter; 1.0204x vs baseline; 1.0204x over previous
import jax
import jax.numpy as jnp
from jax import lax
from jax.experimental import pallas as pl
from jax.experimental.pallas import tpu as pltpu

C = 16


def kernel(x):
    _, m, n = x.shape
    half = m // 2
    ck = half // C

    def body(x_ref, out_ref, send_buf, rs_recv,
             rs_ss, rs_rs, agy_ss, agy_rs, agx_ss, agx_rs, fwd_ss, fwd_rs):
        my_x = lax.axis_index("x")
        my_y = lax.axis_index("y")
        ox = 1 - my_x
        oy = 1 - my_y
        xp = (ox, my_y)
        yp = (my_x, oy)

        r_me = my_x * half
        r_ot = ox * half
        c_me = my_y * n

        barrier = pltpu.get_barrier_semaphore()
        for nbr in (xp, yp):
            pl.semaphore_signal(
                barrier, inc=1, device_id=nbr,
                device_id_type=pl.DeviceIdType.MESH,
            )
        pl.semaphore_wait(barrier, 2)

        rs = []
        for k in range(C):
            send_buf[pl.ds(k * ck, ck), :] = x_ref[
                0, pl.ds(r_ot + k * ck, ck), :
            ].astype(jnp.bfloat16)
            d = pltpu.make_async_remote_copy(
                src_ref=send_buf.at[pl.ds(k * ck, ck), :],
                dst_ref=rs_recv.at[pl.ds(k * ck, ck), :],
                send_sem=rs_ss.at[k],
                recv_sem=rs_rs.at[k],
                device_id=xp,
                device_id_type=pl.DeviceIdType.MESH,
            )
            d.start()
            rs.append(d)

        ag_y, ag_x = [], []
        for k in range(C):
            rs[k].wait_recv()
            out_ref[pl.ds(r_me + k * ck, ck), pl.ds(c_me, n)] = (
                x_ref[0, pl.ds(r_me + k * ck, ck), :].astype(jnp.bfloat16)
                + rs_recv[pl.ds(k * ck, ck), :]
            )
            src = out_ref.at[pl.ds(r_me + k * ck, ck), pl.ds(c_me, n)]
            dx = pltpu.make_async_remote_copy(
                src_ref=src, dst_ref=src,
                send_sem=agx_ss.at[k], recv_sem=agx_rs.at[k],
                device_id=xp, device_id_type=pl.DeviceIdType.MESH,
            )
            dy = pltpu.make_async_remote_copy(
                src_ref=src, dst_ref=src,
                send_sem=agy_ss.at[k], recv_sem=agy_rs.at[k],
                device_id=yp, device_id_type=pl.DeviceIdType.MESH,
            )
            dx.start()
            dy.start()
            ag_x.append(dx)
            ag_y.append(dy)

        fwd = []
        for k in range(C):
            ag_x[k].wait_recv()
            src = out_ref.at[pl.ds(r_ot + k * ck, ck), pl.ds(c_me, n)]
            d = pltpu.make_async_remote_copy(
                src_ref=src, dst_ref=src,
                send_sem=fwd_ss.at[k], recv_sem=fwd_rs.at[k],
                device_id=yp, device_id_type=pl.DeviceIdType.MESH,
            )
            d.start()
            fwd.append(d)

        for k in range(C):
            rs[k].wait_send()
            ag_x[k].wait_send()
            ag_y[k].wait()
            fwd[k].wait()

    return pl.pallas_call(
        body,
        out_shape=jax.ShapeDtypeStruct((m, 2 * n), jnp.bfloat16),
        in_specs=[pl.BlockSpec(memory_space=pltpu.VMEM)],
        out_specs=pl.BlockSpec(memory_space=pltpu.VMEM),
        scratch_shapes=[
            pltpu.VMEM((half, n), jnp.bfloat16),
            pltpu.VMEM((half, n), jnp.bfloat16),
            pltpu.SemaphoreType.DMA((C,)),
            pltpu.SemaphoreType.DMA((C,)),
            pltpu.SemaphoreType.DMA((C,)),
            pltpu.SemaphoreType.DMA((C,)),
            pltpu.SemaphoreType.DMA((C,)),
            pltpu.SemaphoreType.DMA((C,)),
            pltpu.SemaphoreType.DMA((C,)),
            pltpu.SemaphoreType.DMA((C,)),
        ],
        compiler_params=pltpu.CompilerParams(collective_id=0),
    )(x)
